# baseline (speedup 1.0000x reference)
import functools

import jax
import jax.numpy as jnp
from jax import lax
from jax.experimental import pallas as pl
from jax.experimental.pallas import tpu as pltpu
from jax.experimental.pallas import tpu_sc as plsc

NC = 2
NS = 16
NW = NC * NS
LANES = 16
BI = 128


def _build(Bb, Ll, D1, D2, D3):
    DT = D1 + D2 + D3
    NHALF = 5
    LH = Ll // NHALF
    assert LH % 8 == 0 and LH % 2 == 0
    mesh = plsc.VectorSubcoreMesh(core_axis_name="c", subcore_axis_name="s")

    @functools.partial(
        pl.kernel,
        out_type=jax.ShapeDtypeStruct((Ll, DT // 8, Bb // BI, 8, BI),
                                      jnp.float32),
        mesh=mesh,
        compiler_params=pltpu.CompilerParams(use_tc_tiling_on_sc=False,
                                             needs_layout_passes=False),
        scratch_types=[
            pltpu.VMEM((BI, LH), jnp.int32),
            pltpu.VMEM((LH, BI), jnp.int32),
            pltpu.VMEM((LH, BI), jnp.int32),
            pltpu.VMEM((LH, BI), jnp.int32),
            pltpu.VMEM((BI, D1), jnp.float32),
            pltpu.VMEM((BI, D2), jnp.float32),
            pltpu.VMEM((BI, D3), jnp.float32),
            pltpu.VMEM((BI, D1), jnp.float32),
            pltpu.VMEM((BI, D2), jnp.float32),
            pltpu.VMEM((BI, D3), jnp.float32),
            pltpu.VMEM((DT // 8, 8, BI), jnp.float32),
            pltpu.VMEM((DT // 8, 8, BI), jnp.float32),
            pltpu.SemaphoreType.DMA,
            pltpu.SemaphoreType.DMA,
            pltpu.SemaphoreType.DMA,
            pltpu.SemaphoreType.DMA,
        ],
    )
    def k(idx1_hbm, idx2_hbm, idx3_hbm, t1_hbm, t2_hbm, t3_hbm, out_hbm,
          raw_v, it1, it2, it3,
          g1a, g2a, g3a, g1b, g2b, g3b, ca, cb,
          gsem0, gsem1, wsem0, wsem1):
        gbuf = ((g1a, g2a, g3a), (g1b, g2b, g3b))
        comb = (ca, cb)
        gsem = (gsem0, gsem1)
        wsem = (wsem0, wsem1)
        wid = lax.axis_index("s") * NC + lax.axis_index("c")
        base = wid * BI
        iota = lax.iota(jnp.int32, LANES)
        row_ids = [g * LANES + iota for g in range(BI // LANES)]

        def stage_indices(h):
            for idx_hbm, it in ((idx1_hbm, it1), (idx2_hbm, it2),
                                (idx3_hbm, it3)):
                pltpu.sync_copy(
                    idx_hbm.at[pl.ds(base, BI), pl.ds(h * LH, LH)], raw_v)

                def tr_l(lh, carry):
                    col = jnp.broadcast_to(lh, (LANES,)).astype(jnp.int32)
                    for g in range(BI // LANES):
                        v = plsc.load_gather(raw_v, [row_ids[g], col])
                        it[lh, pl.ds(g * LANES, LANES)] = v
                    return carry

                lax.fori_loop(0, LH, tr_l, 0)

        def g_copies(lh, b):
            return tuple(
                pltpu.make_async_copy(t_hbm.at[it.at[lh]], gbuf[b][t], gsem[b])
                for t, (t_hbm, it) in enumerate(
                    ((t1_hbm, it1), (t2_hbm, it2), (t3_hbm, it3))))

        def w_copies(h, lh, b):
            gl = h * LH + lh
            return (pltpu.make_async_copy(
                comb[b], out_hbm.at[gl, :, wid, :, :], wsem[b]),)

        def transpose_rows(b):
            fo_base = 0
            for t, D in enumerate((D1, D2, D3)):
                gb = gbuf[b][t]
                fob = fo_base

                def tr_f(f, carry):
                    fo = fob + f // 8
                    fi = f % 8
                    col = jnp.broadcast_to(f, (LANES,)).astype(jnp.int32)
                    for g in range(BI // LANES):
                        v = plsc.load_gather(gb, [row_ids[g], col])
                        comb[b][fo, fi, pl.ds(g * LANES, LANES)] = v
                    return carry

                lax.fori_loop(0, D, tr_f, 0)
                fo_base += D // 8

        def fire(cps):
            for c in cps:
                c.start()

        def drain(cps):
            for c in cps:
                c.wait()

        for h in range(NHALF):
            stage_indices(h)
            fire(g_copies(0, 0))

            def grp(jj, carry):
                for b in (0, 1):
                    lh = jj * 2 + b
                    drain(g_copies(lh, b))

                    @pl.when(lh + 1 < LH)
                    def _():
                        fire(g_copies(lh + 1, 1 - b))

                    @pl.when(lh >= 2)
                    def _():
                        drain(w_copies(h, lh - 2, b))

                    transpose_rows(b)
                    fire(w_copies(h, lh, b))
                return carry

            lax.fori_loop(0, LH // 2, grp, 0)
            drain(w_copies(h, LH - 2, 0))
            drain(w_copies(h, LH - 1, 1))

    return k


def kernel(sku, category, price, sku_table, cat_table, price_table):
    Bb, Ll = sku.shape
    D1 = sku_table.shape[1]
    D2 = cat_table.shape[1]
    D3 = price_table.shape[1]
    DT = D1 + D2 + D3
    assert Bb % (BI * NW // NW) == 0 and Bb // BI == NW and DT % 8 == 0
    k = _build(Bb, Ll, D1, D2, D3)
    out_phys = k(sku, category, price, sku_table, cat_table, price_table)
    return jnp.transpose(out_phys, (2, 4, 0, 1, 3)).reshape(Bb, Ll, DT)


# speedup vs baseline: 1.2690x; 1.2690x over previous
import functools

import jax
import jax.numpy as jnp
from jax import lax
from jax.experimental import pallas as pl
from jax.experimental.pallas import tpu as pltpu
from jax.experimental.pallas import tpu_sc as plsc

NC = 2
NS = 16
NW = NC * NS
LANES = 16
BI = 128


def _build(Bb, Ll, D1, D2, D3):
    DT = D1 + D2 + D3
    NST = 5
    LH = Ll // NST
    assert LH % 8 == 0 and LH % 2 == 0
    NB = Bb // BI
    mesh = plsc.VectorSubcoreMesh(core_axis_name="c", subcore_axis_name="s")

    @functools.partial(
        pl.kernel,
        out_type=jax.ShapeDtypeStruct((Ll, DT // 8, NB * 8 * BI), jnp.float32),
        mesh=mesh,
        compiler_params=pltpu.CompilerParams(use_tc_tiling_on_sc=False,
                                             needs_layout_passes=False),
        scratch_types=[
            pltpu.VMEM((BI, LH), jnp.int32),
            pltpu.VMEM((LH, BI), jnp.int32),
            pltpu.VMEM((LH, BI), jnp.int32),
            pltpu.VMEM((LH, BI), jnp.int32),
            pltpu.VMEM((BI, D1), jnp.float32),
            pltpu.VMEM((BI, D2), jnp.float32),
            pltpu.VMEM((BI, D3), jnp.float32),
            pltpu.VMEM((BI, D1), jnp.float32),
            pltpu.VMEM((BI, D2), jnp.float32),
            pltpu.VMEM((BI, D3), jnp.float32),
            pltpu.VMEM((DT // 8, 8 * BI), jnp.float32),
            pltpu.VMEM((DT // 8, 8 * BI), jnp.float32),
            pltpu.SemaphoreType.DMA,
            pltpu.SemaphoreType.DMA,
            pltpu.SemaphoreType.DMA,
            pltpu.SemaphoreType.DMA,
        ],
    )
    def k(idx1_hbm, idx2_hbm, idx3_hbm, t1_hbm, t2_hbm, t3_hbm, out_hbm,
          raw_v, it1, it2, it3,
          g1a, g2a, g3a, g1b, g2b, g3b, ca, cb,
          gsem0, gsem1, wsem0, wsem1):
        gbuf = ((g1a, g2a, g3a), (g1b, g2b, g3b))
        comb = (ca, cb)
        gsem = (gsem0, gsem1)
        wsem = (wsem0, wsem1)
        wid = lax.axis_index("s") * NC + lax.axis_index("c")
        base = wid * BI
        iota = lax.iota(jnp.int32, LANES)
        row_ids = [g * LANES + iota for g in range(BI // LANES)]

        def stage_indices(h):
            for idx_hbm, it in ((idx1_hbm, it1), (idx2_hbm, it2),
                                (idx3_hbm, it3)):
                pltpu.sync_copy(
                    idx_hbm.at[pl.ds(base, BI), pl.ds(h * LH, LH)], raw_v)

                @plsc.parallel_loop(0, LH, unroll=2)
                def _(lh):
                    col = jnp.broadcast_to(lh, (LANES,)).astype(jnp.int32)
                    vs = [plsc.load_gather(raw_v, [row_ids[g], col])
                          for g in range(BI // LANES)]
                    for g in range(BI // LANES):
                        it[lh, pl.ds(g * LANES, LANES)] = vs[g]

        def g_copies(lh, b):
            return tuple(
                pltpu.make_async_copy(t_hbm.at[it.at[lh]], gbuf[b][t], gsem[b])
                for t, (t_hbm, it) in enumerate(
                    ((t1_hbm, it1), (t2_hbm, it2), (t3_hbm, it3))))

        def w_copies(h, lh, b):
            gl = h * LH + lh
            return (pltpu.make_async_copy(
                comb[b], out_hbm.at[gl, :, pl.ds(wid * 8 * BI, 8 * BI)],
                wsem[b]),)

        def transpose_rows(b):
            cref = comb[b]
            fo_base = 0
            for t, D in enumerate((D1, D2, D3)):
                gb = gbuf[b][t]
                fob = fo_base

                @plsc.parallel_loop(0, D, unroll=4)
                def _(f):
                    fo = fob + f // 8
                    ofs = (f % 8) * BI
                    col = jnp.broadcast_to(f, (LANES,)).astype(jnp.int32)
                    vs = [plsc.load_gather(gb, [row_ids[g], col])
                          for g in range(BI // LANES)]
                    for g in range(BI // LANES):
                        cref[fo, pl.ds(ofs + g * LANES, LANES)] = vs[g]

                fo_base += D // 8

        def fire(cps):
            for c in cps:
                c.start()

        def drain(cps):
            for c in cps:
                c.wait()

        for h in range(NST):
            stage_indices(h)
            fire(g_copies(0, 0))

            def grp(jj, carry):
                for b in (0, 1):
                    lh = jj * 2 + b
                    drain(g_copies(lh, b))

                    @pl.when(lh + 1 < LH)
                    def _():
                        fire(g_copies(lh + 1, 1 - b))

                    @pl.when(lh >= 2)
                    def _():
                        drain(w_copies(h, lh - 2, b))

                    transpose_rows(b)
                    fire(w_copies(h, lh, b))
                return carry

            lax.fori_loop(0, LH // 2, grp, 0)
            drain(w_copies(h, LH - 2, 0))
            drain(w_copies(h, LH - 1, 1))

    return k


def kernel(sku, category, price, sku_table, cat_table, price_table):
    Bb, Ll = sku.shape
    D1 = sku_table.shape[1]
    D2 = cat_table.shape[1]
    D3 = price_table.shape[1]
    DT = D1 + D2 + D3
    assert Bb % BI == 0 and Bb // BI == NW and DT % 8 == 0
    k = _build(Bb, Ll, D1, D2, D3)
    out_phys = k(sku, category, price, sku_table, cat_table, price_table)
    out5 = out_phys.reshape(Ll, DT // 8, Bb // BI, 8, BI)
    return jnp.transpose(out5, (2, 4, 0, 1, 3)).reshape(Bb, Ll, DT)


# speedup vs baseline: 2.0606x; 1.6238x over previous
import functools

import jax
import jax.numpy as jnp
from jax import lax
from jax.experimental import pallas as pl
from jax.experimental.pallas import tpu as pltpu
from jax.experimental.pallas import tpu_sc as plsc

NC = 2
NS = 16
NW = NC * NS
LANES = 16
BI = 128


def _build(Bb, Ll, D1, D2, D3):
    DT = D1 + D2 + D3
    NST = 5
    LH = Ll // NST
    assert LH % 8 == 0 and LH % 2 == 0
    NB = Bb // BI
    mesh = plsc.VectorSubcoreMesh(core_axis_name="c", subcore_axis_name="s")

    @functools.partial(
        pl.kernel,
        out_type=jax.ShapeDtypeStruct((Ll, DT // 8, NB, 8, BI), jnp.float32),
        mesh=mesh,
        compiler_params=pltpu.CompilerParams(use_tc_tiling_on_sc=False,
                                             needs_layout_passes=False),
        scratch_types=[
            pltpu.VMEM((BI, LH + 1), jnp.int32),
            pltpu.VMEM((LH, BI), jnp.int32),
            pltpu.VMEM((LH, BI), jnp.int32),
            pltpu.VMEM((LH, BI), jnp.int32),
            pltpu.VMEM((BI, D1), jnp.float32),
            pltpu.VMEM((BI, D2), jnp.float32),
            pltpu.VMEM((BI, D3), jnp.float32),
            pltpu.VMEM((BI, D1), jnp.float32),
            pltpu.VMEM((BI, D2), jnp.float32),
            pltpu.VMEM((BI, D3), jnp.float32),
            pltpu.VMEM((DT // 8, 8, BI + 1), jnp.float32),
            pltpu.VMEM((DT // 8, 8, BI + 1), jnp.float32),
            pltpu.SemaphoreType.DMA,
            pltpu.SemaphoreType.DMA,
            pltpu.SemaphoreType.DMA,
            pltpu.SemaphoreType.DMA,
        ],
    )
    def k(idx1_hbm, idx2_hbm, idx3_hbm, t1_hbm, t2_hbm, t3_hbm, out_hbm,
          raw_v, it1, it2, it3,
          g1a, g2a, g3a, g1b, g2b, g3b, ca, cb,
          gsem0, gsem1, wsem0, wsem1):
        gbuf = ((g1a, g2a, g3a), (g1b, g2b, g3b))
        comb = (ca, cb)
        gsem = (gsem0, gsem1)
        wsem = (wsem0, wsem1)
        wid = lax.axis_index("s") * NC + lax.axis_index("c")
        base = wid * BI
        iota = lax.iota(jnp.int32, LANES)
        row_ids = [g * LANES + iota for g in range(BI // LANES)]

        def stage_indices(h):
            for idx_hbm, it in ((idx1_hbm, it1), (idx2_hbm, it2),
                                (idx3_hbm, it3)):
                pltpu.sync_copy(
                    idx_hbm.at[pl.ds(base, BI), pl.ds(h * LH, LH)],
                    raw_v.at[:, pl.ds(0, LH)])

                @plsc.parallel_loop(0, LH, unroll=2)
                def _(lh):
                    col = jnp.broadcast_to(lh, (LANES,)).astype(jnp.int32)
                    vs = [plsc.load_gather(raw_v, [row_ids[g], col])
                          for g in range(BI // LANES)]
                    for g in range(BI // LANES):
                        it[lh, pl.ds(g * LANES, LANES)] = vs[g]

        def g_copies(lh, b):
            return tuple(
                pltpu.make_async_copy(t_hbm.at[it.at[lh]], gbuf[b][t], gsem[b])
                for t, (t_hbm, it) in enumerate(
                    ((t1_hbm, it1), (t2_hbm, it2), (t3_hbm, it3))))

        def w_copies(h, lh, b):
            gl = h * LH + lh
            return (pltpu.make_async_copy(
                comb[b].at[:, :, pl.ds(0, BI)],
                out_hbm.at[gl, :, wid, :, :], wsem[b]),)

        fi_idx = iota % 8
        fof = iota // 8

        def transpose_rows(b):
            cref = comb[b]
            plans = []
            fo_base = 0
            for t, D in enumerate((D1, D2, D3)):
                for f0 in range(0, D, LANES):
                    plans.append((gbuf[b][t], f0, fo_base + f0 // 8 + fof))
                fo_base += D // 8

            @plsc.parallel_loop(0, BI, unroll=2)
            def _(bi):
                bi_vec = jnp.broadcast_to(bi, (LANES,)).astype(jnp.int32)
                vs = [gb[bi, pl.ds(f0, LANES)] for gb, f0, _ in plans]
                for v, (_, _, fo_idx) in zip(vs, plans):
                    plsc.store_scatter(cref, [fo_idx, fi_idx, bi_vec], v)

        def fire(cps):
            for c in cps:
                c.start()

        def drain(cps):
            for c in cps:
                c.wait()

        for h in range(NST):
            stage_indices(h)
            fire(g_copies(0, 0))

            def grp(jj, carry):
                for b in (0, 1):
                    lh = jj * 2 + b
                    drain(g_copies(lh, b))

                    @pl.when(lh + 1 < LH)
                    def _():
                        fire(g_copies(lh + 1, 1 - b))

                    @pl.when(lh >= 2)
                    def _():
                        drain(w_copies(h, lh - 2, b))

                    transpose_rows(b)
                    fire(w_copies(h, lh, b))
                return carry

            lax.fori_loop(0, LH // 2, grp, 0)
            drain(w_copies(h, LH - 2, 0))
            drain(w_copies(h, LH - 1, 1))

    return k


def kernel(sku, category, price, sku_table, cat_table, price_table):
    Bb, Ll = sku.shape
    D1 = sku_table.shape[1]
    D2 = cat_table.shape[1]
    D3 = price_table.shape[1]
    DT = D1 + D2 + D3
    assert Bb % BI == 0 and Bb // BI == NW and DT % 8 == 0
    k = _build(Bb, Ll, D1, D2, D3)
    out_phys = k(sku, category, price, sku_table, cat_table, price_table)
    return jnp.transpose(out_phys, (2, 4, 0, 1, 3)).reshape(Bb, Ll, DT)


# speedup vs baseline: 2.1440x; 1.0405x over previous
import functools

import jax
import jax.numpy as jnp
from jax import lax
from jax.experimental import pallas as pl
from jax.experimental.pallas import tpu as pltpu
from jax.experimental.pallas import tpu_sc as plsc

NC = 2
NS = 16
NW = NC * NS
LANES = 16
BI = 128


def _build(Bb, Ll, D1, D2, D3):
    DT = D1 + D2 + D3
    NST = 5
    LH = Ll // NST
    assert LH % 8 == 0 and LH % 2 == 0
    NB = Bb // BI
    mesh = plsc.VectorSubcoreMesh(core_axis_name="c", subcore_axis_name="s")

    @functools.partial(
        pl.kernel,
        out_type=jax.ShapeDtypeStruct((Ll, DT // 8, NB, 8, BI), jnp.float32),
        mesh=mesh,
        compiler_params=pltpu.CompilerParams(use_tc_tiling_on_sc=False,
                                             needs_layout_passes=False),
        scratch_types=[
            pltpu.VMEM((LH // 8, 8, BI), jnp.int32),
            pltpu.VMEM((LH // 8, 8, BI), jnp.int32),
            pltpu.VMEM((LH // 8, 8, BI), jnp.int32),
            pltpu.VMEM((BI, D1), jnp.float32),
            pltpu.VMEM((BI, D2), jnp.float32),
            pltpu.VMEM((BI, D3), jnp.float32),
            pltpu.VMEM((BI, D1), jnp.float32),
            pltpu.VMEM((BI, D2), jnp.float32),
            pltpu.VMEM((BI, D3), jnp.float32),
            pltpu.VMEM((DT // 8, 8, BI + 1), jnp.float32),
            pltpu.VMEM((DT // 8, 8, BI + 1), jnp.float32),
            pltpu.SemaphoreType.DMA,
            pltpu.SemaphoreType.DMA,
            pltpu.SemaphoreType.DMA,
            pltpu.SemaphoreType.DMA,
        ],
    )
    def k(idx1_hbm, idx2_hbm, idx3_hbm, t1_hbm, t2_hbm, t3_hbm, out_hbm,
          it1, it2, it3,
          g1a, g2a, g3a, g1b, g2b, g3b, ca, cb,
          gsem0, gsem1, wsem0, wsem1):
        gbuf = ((g1a, g2a, g3a), (g1b, g2b, g3b))
        comb = (ca, cb)
        gsem = (gsem0, gsem1)
        wsem = (wsem0, wsem1)
        wid = lax.axis_index("s") * NC + lax.axis_index("c")
        base = wid * BI
        iota = lax.iota(jnp.int32, LANES)
        row_ids = [g * LANES + iota for g in range(BI // LANES)]

        def stage_indices(h):
            for idx_hbm, it in ((idx1_hbm, it1), (idx2_hbm, it2),
                                (idx3_hbm, it3)):
                pltpu.sync_copy(
                    idx_hbm.at[pl.ds(h * (LH // 8), LH // 8), wid], it)

        def g_copies(lh, b):
            return tuple(
                pltpu.make_async_copy(t_hbm.at[it.at[lh // 8, lh % 8]],
                                      gbuf[b][t], gsem[b])
                for t, (t_hbm, it) in enumerate(
                    ((t1_hbm, it1), (t2_hbm, it2), (t3_hbm, it3))))

        def w_copies(h, lh, b):
            gl = h * LH + lh
            return (pltpu.make_async_copy(
                comb[b].at[:, :, pl.ds(0, BI)],
                out_hbm.at[gl, :, wid, :, :], wsem[b]),)

        fi_idx = iota % 8
        fof = iota // 8

        def transpose_rows(b):
            cref = comb[b]
            plans = []
            fo_base = 0
            for t, D in enumerate((D1, D2, D3)):
                for f0 in range(0, D, LANES):
                    plans.append((gbuf[b][t], f0, fo_base + f0 // 8 + fof))
                fo_base += D // 8

            @plsc.parallel_loop(0, BI, unroll=2)
            def _(bi):
                bi_vec = jnp.broadcast_to(bi, (LANES,)).astype(jnp.int32)
                vs = [gb[bi, pl.ds(f0, LANES)] for gb, f0, _ in plans]
                for v, (_, _, fo_idx) in zip(vs, plans):
                    plsc.store_scatter(cref, [fo_idx, fi_idx, bi_vec], v)

        def fire(cps):
            for c in cps:
                c.start()

        def drain(cps):
            for c in cps:
                c.wait()

        for h in range(NST):
            stage_indices(h)
            fire(g_copies(0, 0))

            def grp(jj, carry):
                for b in (0, 1):
                    lh = jj * 2 + b
                    drain(g_copies(lh, b))

                    @pl.when(lh + 1 < LH)
                    def _():
                        fire(g_copies(lh + 1, 1 - b))

                    @pl.when(lh >= 2)
                    def _():
                        drain(w_copies(h, lh - 2, b))

                    transpose_rows(b)
                    fire(w_copies(h, lh, b))
                return carry

            lax.fori_loop(0, LH // 2, grp, 0)
            drain(w_copies(h, LH - 2, 0))
            drain(w_copies(h, LH - 1, 1))

    return k


def kernel(sku, category, price, sku_table, cat_table, price_table):
    Bb, Ll = sku.shape
    NBT, NLT = Bb // BI, Ll // 8
    D1 = sku_table.shape[1]
    D2 = cat_table.shape[1]
    D3 = price_table.shape[1]
    DT = D1 + D2 + D3
    assert Bb % BI == 0 and Bb // BI == NW and DT % 8 == 0
    k = _build(Bb, Ll, D1, D2, D3)

    def to_lmajor(x):
        return x.reshape(NBT, BI, NLT, 8).transpose(2, 0, 3, 1)

    out_phys = k(to_lmajor(sku), to_lmajor(category), to_lmajor(price),
                 sku_table, cat_table, price_table)
    return jnp.transpose(out_phys, (2, 4, 0, 1, 3)).reshape(Bb, Ll, DT)


# speedup vs baseline: 2.7206x; 1.2690x over previous
import functools

import jax
import jax.numpy as jnp
from jax import lax
from jax.experimental import pallas as pl
from jax.experimental.pallas import tpu as pltpu
from jax.experimental.pallas import tpu_sc as plsc

NC = 2
NS = 16
NW = NC * NS
LANES = 16
BI = 128


def _build(Bb, Ll, D1, D2, D3, ct_rows, pt_rows):
    DT = D1 + D2 + D3
    NST = 5
    LH = Ll // NST
    assert LH % 8 == 0 and LH % 2 == 0
    NB = Bb // BI
    mesh = plsc.VectorSubcoreMesh(core_axis_name="c", subcore_axis_name="s")

    @functools.partial(
        pl.kernel,
        out_type=jax.ShapeDtypeStruct((Ll, DT // 8, NB, 8, BI), jnp.float32),
        mesh=mesh,
        compiler_params=pltpu.CompilerParams(use_tc_tiling_on_sc=False,
                                             needs_layout_passes=False),
        scratch_types=[
            pltpu.VMEM((LH // 8, 8, BI), jnp.int32),
            pltpu.VMEM((LH // 8, 8, BI), jnp.int32),
            pltpu.VMEM((LH // 8, 8, BI), jnp.int32),
            pltpu.VMEM((BI, D1), jnp.float32),
            pltpu.VMEM((BI, D1), jnp.float32),
            pltpu.VMEM((ct_rows, 128), jnp.float32),
            pltpu.VMEM((pt_rows, 128), jnp.float32),

            pltpu.VMEM((DT // 8, 8, BI + 1), jnp.float32),
            pltpu.VMEM((DT // 8, 8, BI + 1), jnp.float32),
            pltpu.SemaphoreType.DMA,
            pltpu.SemaphoreType.DMA,
            pltpu.SemaphoreType.DMA,
            pltpu.SemaphoreType.DMA,
        ],
    )
    def k(idx1_hbm, idx2_hbm, idx3_hbm, t1_hbm, t2_hbm, t3_hbm, out_hbm,
          it1, it2, it3,
          g1a, g1b, ct, pt, ca, cb,
          gsem0, gsem1, wsem0, wsem1):
        gbuf = (g1a, g1b)
        comb = (ca, cb)
        gsem = (gsem0, gsem1)
        wsem = (wsem0, wsem1)
        wid = lax.axis_index("s") * NC + lax.axis_index("c")
        base = wid * BI
        iota = lax.iota(jnp.int32, LANES)
        row_ids = [g * LANES + iota for g in range(BI // LANES)]

        def stage_indices(h):
            for idx_hbm, it in ((idx1_hbm, it1), (idx2_hbm, it2),
                                (idx3_hbm, it3)):
                pltpu.sync_copy(
                    idx_hbm.at[pl.ds(h * (LH // 8), LH // 8), wid], it)

        def g_copies(lh, b):
            return (pltpu.make_async_copy(
                t1_hbm.at[it1.at[lh // 8, lh % 8]], gbuf[b], gsem[b]),)

        def w_copies(h, lh, b):
            gl = h * LH + lh
            return (pltpu.make_async_copy(
                comb[b].at[:, :, pl.ds(0, BI)],
                out_hbm.at[gl, :, wid, :, :], wsem[b]),)

        fi_idx = iota % 8
        fof = iota // 8

        def transpose_rows(lh, b):
            cref = comb[b]
            plans = [(f0, f0 // 8 + fof) for f0 in range(0, D1, LANES)]

            @plsc.parallel_loop(0, BI, unroll=2)
            def _(bi):
                bi_vec = jnp.broadcast_to(bi, (LANES,)).astype(jnp.int32)
                vs = [gbuf[b][bi, pl.ds(f0, LANES)] for f0, _ in plans]
                for v, (_, fo_idx) in zip(vs, plans):
                    plsc.store_scatter(cref, [fo_idx, fi_idx, bi_vec], v)

            lt = lh // 8
            li = lh % 8

            @plsc.parallel_loop(0, BI // LANES, unroll=1)
            def _(g):
                r2 = it2[lt, li, pl.ds(g * LANES, LANES)]
                row2 = (r2 >> 7) * 8
                col2 = r2 & 127
                r3 = it3[lt, li, pl.ds(g * LANES, LANES)]
                for f in range(D2):
                    rv = row2 + (f // 8 * 64 + f % 8)
                    v = plsc.load_gather(ct, [rv, col2])
                    cref[D1 // 8 + f // 8, f % 8,
                         pl.ds(g * LANES, LANES)] = v
                for f in range(D3):
                    rv = jnp.broadcast_to(f // 8 * 8 + f % 8,
                                          (LANES,)).astype(jnp.int32)
                    v = plsc.load_gather(pt, [rv, r3])
                    cref[(D1 + D2) // 8 + f // 8, f % 8,
                         pl.ds(g * LANES, LANES)] = v

        def fire(cps):
            for c in cps:
                c.start()

        def drain(cps):
            for c in cps:
                c.wait()

        pltpu.sync_copy(t2_hbm, ct)
        pltpu.sync_copy(t3_hbm, pt)
        for h in range(NST):
            stage_indices(h)
            fire(g_copies(0, 0))

            def grp(jj, carry):
                for b in (0, 1):
                    lh = jj * 2 + b
                    drain(g_copies(lh, b))

                    @pl.when(lh + 1 < LH)
                    def _():
                        fire(g_copies(lh + 1, 1 - b))

                    @pl.when(lh >= 2)
                    def _():
                        drain(w_copies(h, lh - 2, b))

                    transpose_rows(lh, b)
                    fire(w_copies(h, lh, b))
                return carry

            lax.fori_loop(0, LH // 2, grp, 0)
            drain(w_copies(h, LH - 2, 0))
            drain(w_copies(h, LH - 1, 1))

    return k


def kernel(sku, category, price, sku_table, cat_table, price_table):
    Bb, Ll = sku.shape
    NBT, NLT = Bb // BI, Ll // 8
    D1 = sku_table.shape[1]
    D2 = cat_table.shape[1]
    D3 = price_table.shape[1]
    DT = D1 + D2 + D3
    assert Bb % BI == 0 and Bb // BI == NW and DT % 8 == 0

    def table_phys(t):
        vp = -t.shape[0] % 128
        d = t.shape[1]
        tp = jnp.pad(t, ((0, vp), (0, 0)))
        return (tp.reshape(-1, 128, d // 8, 8).transpose(2, 0, 3, 1)
                .reshape(-1, 128))

    def to_lmajor(x):
        return x.reshape(NBT, BI, NLT, 8).transpose(2, 0, 3, 1)

    ctp = table_phys(cat_table)
    ptp = table_phys(price_table)
    k = _build(Bb, Ll, D1, D2, D3, ctp.shape[0], ptp.shape[0])
    out_phys = k(to_lmajor(sku), to_lmajor(category), to_lmajor(price),
                 sku_table, ctp, ptp)
    return jnp.transpose(out_phys, (2, 4, 0, 1, 3)).reshape(Bb, Ll, DT)
